# baseline (speedup 1.0000x reference)
import functools

import jax
import jax.numpy as jnp
from jax import lax
from jax.experimental import pallas as pl
from jax.experimental.pallas import tpu as pltpu
from jax.experimental.pallas import tpu_sc as plsc

NN = 10000
CH_F = 128
NP = 10240
CHUNK = 128
NCHW = 80
NHALF = 40
NB = 2
ND = 4
EW = CHUNK * NCHW
EPAD = EW * 32
RPT = NP // 16

_MESH = plsc.VectorSubcoreMesh(core_axis_name="c", subcore_axis_name="s")


def _deg_body(dstr, z1, out, idxd, ones, accd, dsem):
    c = lax.axis_index("c")
    s = lax.axis_index("s")
    wid = c * 16 + s
    pltpu.sync_copy(z1, accd.at[pl.ds(s * RPT, RPT)])
    pltpu.sync_copy(dstr.at[wid], idxd)
    for i in range(CHUNK // 16):
        ones[pl.ds(i * 16, 16)] = jnp.ones((16,), jnp.float32)
    plsc.subcore_barrier()

    for b in range(ND):
        pltpu.async_copy(ones, accd.at[idxd.at[b]], dsem, add=True)

    def step(j, carry):
        pltpu.make_async_copy(ones, accd.at[idxd.at[j]], dsem).wait()
        nj = j + ND

        @pl.when(nj < NCHW)
        def _():
            pltpu.async_copy(ones, accd.at[idxd.at[nj]], dsem, add=True)

        return carry

    lax.fori_loop(0, NCHW, step, 0)
    plsc.subcore_barrier()
    pltpu.sync_copy(accd.at[pl.ds(s * RPT, RPT)], out.at[pl.ds(c * NP + s * RPT, RPT)])


_deg = functools.partial(
    pl.kernel,
    out_type=jax.ShapeDtypeStruct((2 * NP,), jnp.float32),
    mesh=_MESH,
    scratch_types=[
        pltpu.VMEM((NCHW, CHUNK), jnp.int32),
        pltpu.VMEM((CHUNK,), jnp.float32),
        pltpu.VMEM_SHARED((NP,), jnp.float32),
        pltpu.SemaphoreType.DMA,
    ],
)(_deg_body)


def _prop_body(y, srcr, dstr, zrows, out, idxs, idxd, rows, acc, gsem, ssem):
    c = lax.axis_index("c")
    s = lax.axis_index("s")
    wid = c * 16 + s
    pltpu.sync_copy(zrows, acc.at[pl.ds(s * RPT, RPT)])
    plsc.subcore_barrier()

    for h in range(NCHW // NHALF):
        pltpu.sync_copy(srcr.at[wid, pl.ds(h * NHALF, NHALF)], idxs)
        pltpu.sync_copy(dstr.at[wid, pl.ds(h * NHALF, NHALF)], idxd)

        for b in range(NB):
            pltpu.async_copy(y.at[idxs.at[b]], rows.at[b], gsem)

        def step(j, carry):
            b = lax.rem(j, NB)
            pltpu.make_async_copy(y.at[idxs.at[j]], rows.at[b], gsem).wait()
            pltpu.async_copy(rows.at[b], acc.at[idxd.at[j]], ssem, add=True)
            pltpu.make_async_copy(rows.at[b], acc.at[idxd.at[j]], ssem).wait()
            nj = j + NB

            @pl.when(nj < NHALF)
            def _():
                pltpu.async_copy(y.at[idxs.at[nj]], rows.at[b], gsem)

            return carry

        lax.fori_loop(0, NHALF, step, 0)

    plsc.subcore_barrier()
    pltpu.sync_copy(acc.at[pl.ds(s * RPT, RPT)], out.at[c, pl.ds(s * RPT, RPT)])


_prop = functools.partial(
    pl.kernel,
    out_type=jax.ShapeDtypeStruct((2, NP, CH_F), jnp.float32),
    mesh=_MESH,
    compiler_params=pltpu.CompilerParams(use_tc_tiling_on_sc=False),
    scratch_types=[
        pltpu.VMEM((NHALF, CHUNK), jnp.int32),
        pltpu.VMEM((NHALF, CHUNK), jnp.int32),
        pltpu.VMEM((NB, CHUNK, CH_F), jnp.float32),
        pltpu.VMEM_SHARED((NP, CH_F), jnp.float32),
        pltpu.SemaphoreType.DMA,
        pltpu.SemaphoreType.DMA,
    ],
)(_prop_body)


BR = 1000


def _prescale_body(d0, d1, x, xp, inv):
    d = d0[...] + d1[...] + 1.0
    r = lax.rsqrt(d)
    xp[...] = x[...] * r
    inv[...] = r


_prescale = pl.pallas_call(
    _prescale_body,
    grid=(NN // BR,),
    in_specs=[
        pl.BlockSpec((BR, 1), lambda i: (i, 0)),
        pl.BlockSpec((BR, 1), lambda i: (i, 0)),
        pl.BlockSpec((BR, CH_F), lambda i: (i, 0)),
    ],
    out_specs=[
        pl.BlockSpec((BR, CH_F), lambda i: (i, 0)),
        pl.BlockSpec((BR, 1), lambda i: (i, 0)),
    ],
    out_shape=[
        jax.ShapeDtypeStruct((NN, CH_F), jnp.float32),
        jax.ShapeDtypeStruct((NN, 1), jnp.float32),
    ],
)


def _dense_body(pa, pb, xp, inv, w1, b1, w2, out):
    z1 = inv[...] * (pa[0] + pb[0] + xp[...])
    h = jnp.dot(z1, w1[...], preferred_element_type=jnp.float32) + b1[...]
    h = jnp.maximum(h, 0.0)
    out[...] = jnp.dot(h, w2[...], preferred_element_type=jnp.float32) * inv[...]


_dense = pl.pallas_call(
    _dense_body,
    grid=(NN // BR,),
    in_specs=[
        pl.BlockSpec((1, BR, CH_F), lambda i: (0, i, 0)),
        pl.BlockSpec((1, BR, CH_F), lambda i: (1, i, 0)),
        pl.BlockSpec((BR, CH_F), lambda i: (i, 0)),
        pl.BlockSpec((BR, 1), lambda i: (i, 0)),
        pl.BlockSpec((CH_F, 2 * CH_F), lambda i: (0, 0)),
        pl.BlockSpec((1, 2 * CH_F), lambda i: (0, 0)),
        pl.BlockSpec((2 * CH_F, CH_F), lambda i: (0, 0)),
    ],
    out_specs=pl.BlockSpec((BR, CH_F), lambda i: (i, 0)),
    out_shape=jax.ShapeDtypeStruct((NN, CH_F), jnp.float32),
)


def _final_body(pa, pb, z2p, inv, b2, out):
    out[...] = inv[...] * (pa[0] + pb[0] + z2p[...]) + b2[...]


_final = pl.pallas_call(
    _final_body,
    grid=(NN // BR,),
    in_specs=[
        pl.BlockSpec((1, BR, CH_F), lambda i: (0, i, 0)),
        pl.BlockSpec((1, BR, CH_F), lambda i: (1, i, 0)),
        pl.BlockSpec((BR, CH_F), lambda i: (i, 0)),
        pl.BlockSpec((BR, 1), lambda i: (i, 0)),
        pl.BlockSpec((1, CH_F), lambda i: (0, 0)),
    ],
    out_specs=pl.BlockSpec((BR, CH_F), lambda i: (i, 0)),
    out_shape=jax.ShapeDtypeStruct((NN, CH_F), jnp.float32),
)


def kernel(x, edge_index, W1, b1, W2, b2):
    ei = edge_index.astype(jnp.int32)
    npad = EPAD - ei.shape[1]
    src = jnp.concatenate([ei[0], jnp.zeros((npad,), jnp.int32)])
    dst = jnp.concatenate([ei[1], jnp.full((npad,), NN, jnp.int32)])
    srcr = src.reshape(32, NCHW, CHUNK)
    dstr = dst.reshape(32, NCHW, CHUNK)
    zrows = jnp.zeros((RPT, CH_F), jnp.float32)
    z1 = jnp.zeros((RPT,), jnp.float32)

    degp = _deg(dstr, z1)
    d0 = degp[:NP].reshape(NP, 1)
    d1 = degp[NP:].reshape(NP, 1)
    xp, inv = _prescale(d0, d1, x)
    p1 = _prop(xp, srcr, dstr, zrows)
    z2p = _dense(p1, p1, xp, inv, W1, b1.reshape(1, -1), W2)
    p2 = _prop(z2p, srcr, dstr, zrows)
    out = _final(p2, p2, z2p, inv, b2.reshape(1, -1))
    return out


# speedup vs baseline: 2.3313x; 2.3313x over previous
import functools

import jax
import jax.numpy as jnp
from jax import lax
from jax.experimental import pallas as pl
from jax.experimental.pallas import tpu as pltpu
from jax.experimental.pallas import tpu_sc as plsc

NN = 10000
CH_F = 128
CHH = 64
NP = 10240
CHUNK = 128
NCHW = 160
NQ = 4
NCQ = NCHW // NQ
NB = 4
ND = 4
NCHD = 80
EW = CHUNK * NCHW
EPAD = EW * 16
RPT = NP // 16
RST = NN // 16

_MESH = plsc.VectorSubcoreMesh(core_axis_name="c", subcore_axis_name="s")


def _deg_body(dstr, z1, out, idxd, ones, accd, dsem):
    c = lax.axis_index("c")
    s = lax.axis_index("s")
    wid = c * 16 + s
    pltpu.sync_copy(z1, accd.at[pl.ds(s * RPT, RPT)])
    pltpu.sync_copy(dstr.at[wid], idxd)
    for i in range(CHUNK // 16):
        ones[pl.ds(i * 16, 16)] = jnp.ones((16,), jnp.float32)
    plsc.subcore_barrier()

    for b in range(ND):
        pltpu.async_copy(ones, accd.at[idxd.at[b]], dsem, add=True)

    def step(j, carry):
        pltpu.make_async_copy(ones, accd.at[idxd.at[j]], dsem).wait()
        nj = j + ND

        @pl.when(nj < NCHD)
        def _():
            pltpu.async_copy(ones, accd.at[idxd.at[nj]], dsem, add=True)

        return carry

    lax.fori_loop(0, NCHD, step, 0)
    plsc.subcore_barrier()
    pltpu.sync_copy(accd.at[pl.ds(s * RPT, RPT)], out.at[pl.ds(c * NP + s * RPT, RPT)])


_deg = functools.partial(
    pl.kernel,
    out_type=jax.ShapeDtypeStruct((2 * NP,), jnp.float32),
    mesh=_MESH,
    scratch_types=[
        pltpu.VMEM((NCHD, CHUNK), jnp.int32),
        pltpu.VMEM((CHUNK,), jnp.float32),
        pltpu.VMEM_SHARED((NP,), jnp.float32),
        pltpu.SemaphoreType.DMA,
    ],
)(_deg_body)


def _prop_body(y, srcr, dstr, zrows, out, idxs, idxd, rows, ytab, acc, gsem, ssem):
    c = lax.axis_index("c")
    s = lax.axis_index("s")
    pltpu.sync_copy(zrows, acc.at[pl.ds(s * RPT, RPT)])
    pltpu.sync_copy(y.at[pl.ds(c * NN + s * RST, RST)], ytab.at[pl.ds(s * RST, RST)])
    plsc.subcore_barrier()

    for q in range(NQ):
        pltpu.sync_copy(srcr.at[s, pl.ds(q * NCQ, NCQ)], idxs)
        pltpu.sync_copy(dstr.at[s, pl.ds(q * NCQ, NCQ)], idxd)

        for b in range(NB):
            pltpu.async_copy(ytab.at[idxs.at[b]], rows.at[b], gsem)

        def step(j, carry):
            b = lax.rem(j, NB)
            pltpu.make_async_copy(ytab.at[idxs.at[j]], rows.at[b], gsem).wait()
            pltpu.async_copy(rows.at[b], acc.at[idxd.at[j]], ssem, add=True)
            pltpu.make_async_copy(rows.at[b], acc.at[idxd.at[j]], ssem).wait()
            nj = j + NB

            @pl.when(nj < NCQ)
            def _():
                pltpu.async_copy(ytab.at[idxs.at[nj]], rows.at[b], gsem)

            return carry

        lax.fori_loop(0, NCQ, step, 0)

    plsc.subcore_barrier()
    pltpu.sync_copy(acc.at[pl.ds(s * RPT, RPT)], out.at[c, pl.ds(s * RPT, RPT)])


_prop = functools.partial(
    pl.kernel,
    out_type=jax.ShapeDtypeStruct((2, NP, CHH), jnp.float32),
    mesh=_MESH,
    compiler_params=pltpu.CompilerParams(use_tc_tiling_on_sc=False),
    scratch_types=[
        pltpu.VMEM((NCQ, CHUNK), jnp.int32),
        pltpu.VMEM((NCQ, CHUNK), jnp.int32),
        pltpu.VMEM((NB, CHUNK, CHH), jnp.float32),
        pltpu.VMEM_SHARED((NN, CHH), jnp.float32),
        pltpu.VMEM_SHARED((NP, CHH), jnp.float32),
        pltpu.SemaphoreType.DMA,
        pltpu.SemaphoreType.DMA,
    ],
)(_prop_body)


BR = 1000


def _prescale_body(d0, d1, x, xp, inv):
    d = d0[...] + d1[...] + 1.0
    r = lax.rsqrt(d)
    v = x[...] * r
    xp[0] = v[:, :CHH]
    xp[1] = v[:, CHH:]
    inv[...] = r


_prescale = pl.pallas_call(
    _prescale_body,
    grid=(NN // BR,),
    in_specs=[
        pl.BlockSpec((BR, 1), lambda i: (i, 0)),
        pl.BlockSpec((BR, 1), lambda i: (i, 0)),
        pl.BlockSpec((BR, CH_F), lambda i: (i, 0)),
    ],
    out_specs=[
        pl.BlockSpec((2, BR, CHH), lambda i: (0, i, 0)),
        pl.BlockSpec((BR, 1), lambda i: (i, 0)),
    ],
    out_shape=[
        jax.ShapeDtypeStruct((2, NN, CHH), jnp.float32),
        jax.ShapeDtypeStruct((NN, 1), jnp.float32),
    ],
)


def _dense_body(pa, pb, xa, xb, inv, w1, b1, w2, out):
    p = jnp.concatenate([pa[0], pb[0]], axis=1)
    xpv = jnp.concatenate([xa[0], xb[0]], axis=1)
    z1 = inv[...] * (p + xpv)
    h = jnp.dot(z1, w1[...], preferred_element_type=jnp.float32) + b1[...]
    h = jnp.maximum(h, 0.0)
    z2 = jnp.dot(h, w2[...], preferred_element_type=jnp.float32) * inv[...]
    out[0] = z2[:, :CHH]
    out[1] = z2[:, CHH:]


_dense = pl.pallas_call(
    _dense_body,
    grid=(NN // BR,),
    in_specs=[
        pl.BlockSpec((1, BR, CHH), lambda i: (0, i, 0)),
        pl.BlockSpec((1, BR, CHH), lambda i: (1, i, 0)),
        pl.BlockSpec((1, BR, CHH), lambda i: (0, i, 0)),
        pl.BlockSpec((1, BR, CHH), lambda i: (1, i, 0)),
        pl.BlockSpec((BR, 1), lambda i: (i, 0)),
        pl.BlockSpec((CH_F, 2 * CH_F), lambda i: (0, 0)),
        pl.BlockSpec((1, 2 * CH_F), lambda i: (0, 0)),
        pl.BlockSpec((2 * CH_F, CH_F), lambda i: (0, 0)),
    ],
    out_specs=pl.BlockSpec((2, BR, CHH), lambda i: (0, i, 0)),
    out_shape=jax.ShapeDtypeStruct((2, NN, CHH), jnp.float32),
)


def _final_body(pa, pb, za, zb, inv, b2, out):
    p = jnp.concatenate([pa[0], pb[0]], axis=1)
    z = jnp.concatenate([za[0], zb[0]], axis=1)
    out[...] = inv[...] * (p + z) + b2[...]


_final = pl.pallas_call(
    _final_body,
    grid=(NN // BR,),
    in_specs=[
        pl.BlockSpec((1, BR, CHH), lambda i: (0, i, 0)),
        pl.BlockSpec((1, BR, CHH), lambda i: (1, i, 0)),
        pl.BlockSpec((1, BR, CHH), lambda i: (0, i, 0)),
        pl.BlockSpec((1, BR, CHH), lambda i: (1, i, 0)),
        pl.BlockSpec((BR, 1), lambda i: (i, 0)),
        pl.BlockSpec((1, CH_F), lambda i: (0, 0)),
    ],
    out_specs=pl.BlockSpec((BR, CH_F), lambda i: (i, 0)),
    out_shape=jax.ShapeDtypeStruct((NN, CH_F), jnp.float32),
)


def kernel(x, edge_index, W1, b1, W2, b2):
    ei = edge_index.astype(jnp.int32)
    npad = EPAD - ei.shape[1]
    src = jnp.concatenate([ei[0], jnp.zeros((npad,), jnp.int32)])
    dst = jnp.concatenate([ei[1], jnp.full((npad,), NN, jnp.int32)])
    srcr = src.reshape(16, NCHW, CHUNK)
    dstr = dst.reshape(16, NCHW, CHUNK)
    dstr32 = dst.reshape(32, NCHD, CHUNK)
    zrows = jnp.zeros((RPT, CHH), jnp.float32)
    z1 = jnp.zeros((RPT,), jnp.float32)

    degp = _deg(dstr32, z1)
    d0 = degp[:NP].reshape(NP, 1)
    d1 = degp[NP:].reshape(NP, 1)
    xp, inv = _prescale(d0, d1, x)
    xp2 = xp.reshape(2 * NN, CHH)
    p1 = _prop(xp2, srcr, dstr, zrows)
    z2p = _dense(p1, p1, xp, xp, inv, W1, b1.reshape(1, -1), W2)
    p2 = _prop(z2p.reshape(2 * NN, CHH), srcr, dstr, zrows)
    out = _final(p2, p2, z2p, z2p, inv, b2.reshape(1, -1))
    return out


# speedup vs baseline: 2.5481x; 1.0930x over previous
import functools

import jax
import jax.numpy as jnp
from jax import lax
from jax.experimental import pallas as pl
from jax.experimental.pallas import tpu as pltpu
from jax.experimental.pallas import tpu_sc as plsc

NN = 10000
CH_F = 128
CHH = 64
NP = 10240
CHUNK = 128
NCHW = 160
NQ = 8
NCQ = NCHW // NQ
NB = 5
GA = 2
SD = NB - GA
ND = 4
NCHD = 80
EW = CHUNK * NCHW
EPAD = EW * 16
RPT = NP // 16
RST = NN // 16

_MESH = plsc.VectorSubcoreMesh(core_axis_name="c", subcore_axis_name="s")


def _deg_body(dstr, z1, out, idxd, ones, accd, dsem):
    c = lax.axis_index("c")
    s = lax.axis_index("s")
    wid = c * 16 + s
    pltpu.sync_copy(z1, accd.at[pl.ds(s * RPT, RPT)])
    pltpu.sync_copy(dstr.at[wid], idxd)
    for i in range(CHUNK // 16):
        ones[pl.ds(i * 16, 16)] = jnp.ones((16,), jnp.float32)
    plsc.subcore_barrier()

    for b in range(ND):
        pltpu.async_copy(ones, accd.at[idxd.at[b]], dsem, add=True)

    def step(j, carry):
        pltpu.make_async_copy(ones, accd.at[idxd.at[j]], dsem).wait()
        nj = j + ND

        @pl.when(nj < NCHD)
        def _():
            pltpu.async_copy(ones, accd.at[idxd.at[nj]], dsem, add=True)

        return carry

    lax.fori_loop(0, NCHD, step, 0)
    plsc.subcore_barrier()
    pltpu.sync_copy(accd.at[pl.ds(s * RPT, RPT)], out.at[pl.ds(c * NP + s * RPT, RPT)])


_deg = functools.partial(
    pl.kernel,
    out_type=jax.ShapeDtypeStruct((2 * NP,), jnp.float32),
    mesh=_MESH,
    scratch_types=[
        pltpu.VMEM((NCHD, CHUNK), jnp.int32),
        pltpu.VMEM((CHUNK,), jnp.float32),
        pltpu.VMEM_SHARED((NP,), jnp.float32),
        pltpu.SemaphoreType.DMA,
    ],
)(_deg_body)


def _prop_body(y, srcr, dstr, zrows, out, idxs, idxd, rows, ytab, acc, gsem, ssem):
    c = lax.axis_index("c")
    s = lax.axis_index("s")
    pltpu.sync_copy(zrows, acc.at[pl.ds(s * RPT, RPT)])
    pltpu.sync_copy(y.at[pl.ds(c * NN + s * RST, RST)], ytab.at[pl.ds(s * RST, RST)])
    plsc.subcore_barrier()

    for q in range(NQ):
        pltpu.sync_copy(srcr.at[s, pl.ds(q * NCQ, NCQ)], idxs)
        pltpu.sync_copy(dstr.at[s, pl.ds(q * NCQ, NCQ)], idxd)

        for b in range(GA):
            pltpu.async_copy(ytab.at[idxs.at[b]], rows.at[b], gsem)

        def step(j, carry):
            b = lax.rem(j, NB)
            pltpu.make_async_copy(ytab.at[idxs.at[j]], rows.at[b], gsem).wait()
            pltpu.async_copy(rows.at[b], acc.at[idxd.at[j]], ssem, add=True)

            @pl.when(j >= SD)
            def _():
                pltpu.make_async_copy(rows.at[0], acc.at[idxd.at[0]], ssem).wait()

            nj = j + GA

            @pl.when(nj < NCQ)
            def _():
                pltpu.async_copy(ytab.at[idxs.at[nj]], rows.at[lax.rem(nj, NB)], gsem)

            return carry

        lax.fori_loop(0, NCQ, step, 0)

        def drain(j, carry):
            pltpu.make_async_copy(rows.at[0], acc.at[idxd.at[0]], ssem).wait()
            return carry

        lax.fori_loop(0, SD, drain, 0)

    plsc.subcore_barrier()
    pltpu.sync_copy(acc.at[pl.ds(s * RPT, RPT)], out.at[c, pl.ds(s * RPT, RPT)])


_prop = functools.partial(
    pl.kernel,
    out_type=jax.ShapeDtypeStruct((2, NP, CHH), jnp.float32),
    mesh=_MESH,
    compiler_params=pltpu.CompilerParams(use_tc_tiling_on_sc=False),
    scratch_types=[
        pltpu.VMEM((NCQ, CHUNK), jnp.int32),
        pltpu.VMEM((NCQ, CHUNK), jnp.int32),
        pltpu.VMEM((NB, CHUNK, CHH), jnp.float32),
        pltpu.VMEM_SHARED((NN, CHH), jnp.float32),
        pltpu.VMEM_SHARED((NP, CHH), jnp.float32),
        pltpu.SemaphoreType.DMA,
        pltpu.SemaphoreType.DMA,
    ],
)(_prop_body)


BR = 1000


def _prescale_body(d0, d1, x, xp, inv):
    d = d0[...] + d1[...] + 1.0
    r = lax.rsqrt(d)
    v = x[...] * r
    xp[0] = v[:, :CHH]
    xp[1] = v[:, CHH:]
    inv[...] = r


_prescale = pl.pallas_call(
    _prescale_body,
    grid=(NN // BR,),
    in_specs=[
        pl.BlockSpec((BR, 1), lambda i: (i, 0)),
        pl.BlockSpec((BR, 1), lambda i: (i, 0)),
        pl.BlockSpec((BR, CH_F), lambda i: (i, 0)),
    ],
    out_specs=[
        pl.BlockSpec((2, BR, CHH), lambda i: (0, i, 0)),
        pl.BlockSpec((BR, 1), lambda i: (i, 0)),
    ],
    out_shape=[
        jax.ShapeDtypeStruct((2, NN, CHH), jnp.float32),
        jax.ShapeDtypeStruct((NN, 1), jnp.float32),
    ],
)


def _dense_body(pa, pb, xa, xb, inv, w1, b1, w2, out):
    p = jnp.concatenate([pa[0], pb[0]], axis=1)
    xpv = jnp.concatenate([xa[0], xb[0]], axis=1)
    z1 = inv[...] * (p + xpv)
    h = jnp.dot(z1, w1[...], preferred_element_type=jnp.float32) + b1[...]
    h = jnp.maximum(h, 0.0)
    z2 = jnp.dot(h, w2[...], preferred_element_type=jnp.float32) * inv[...]
    out[0] = z2[:, :CHH]
    out[1] = z2[:, CHH:]


_dense = pl.pallas_call(
    _dense_body,
    grid=(NN // BR,),
    in_specs=[
        pl.BlockSpec((1, BR, CHH), lambda i: (0, i, 0)),
        pl.BlockSpec((1, BR, CHH), lambda i: (1, i, 0)),
        pl.BlockSpec((1, BR, CHH), lambda i: (0, i, 0)),
        pl.BlockSpec((1, BR, CHH), lambda i: (1, i, 0)),
        pl.BlockSpec((BR, 1), lambda i: (i, 0)),
        pl.BlockSpec((CH_F, 2 * CH_F), lambda i: (0, 0)),
        pl.BlockSpec((1, 2 * CH_F), lambda i: (0, 0)),
        pl.BlockSpec((2 * CH_F, CH_F), lambda i: (0, 0)),
    ],
    out_specs=pl.BlockSpec((2, BR, CHH), lambda i: (0, i, 0)),
    out_shape=jax.ShapeDtypeStruct((2, NN, CHH), jnp.float32),
)


def _final_body(pa, pb, za, zb, inv, b2, out):
    p = jnp.concatenate([pa[0], pb[0]], axis=1)
    z = jnp.concatenate([za[0], zb[0]], axis=1)
    out[...] = inv[...] * (p + z) + b2[...]


_final = pl.pallas_call(
    _final_body,
    grid=(NN // BR,),
    in_specs=[
        pl.BlockSpec((1, BR, CHH), lambda i: (0, i, 0)),
        pl.BlockSpec((1, BR, CHH), lambda i: (1, i, 0)),
        pl.BlockSpec((1, BR, CHH), lambda i: (0, i, 0)),
        pl.BlockSpec((1, BR, CHH), lambda i: (1, i, 0)),
        pl.BlockSpec((BR, 1), lambda i: (i, 0)),
        pl.BlockSpec((1, CH_F), lambda i: (0, 0)),
    ],
    out_specs=pl.BlockSpec((BR, CH_F), lambda i: (i, 0)),
    out_shape=jax.ShapeDtypeStruct((NN, CH_F), jnp.float32),
)


def kernel(x, edge_index, W1, b1, W2, b2):
    ei = edge_index.astype(jnp.int32)
    npad = EPAD - ei.shape[1]
    src = jnp.concatenate([ei[0], jnp.zeros((npad,), jnp.int32)])
    dst = jnp.concatenate([ei[1], jnp.full((npad,), NN, jnp.int32)])
    srcr = src.reshape(16, NCHW, CHUNK)
    dstr = dst.reshape(16, NCHW, CHUNK)
    dstr32 = dst.reshape(32, NCHD, CHUNK)
    zrows = jnp.zeros((RPT, CHH), jnp.float32)
    z1 = jnp.zeros((RPT,), jnp.float32)

    degp = _deg(dstr32, z1)
    d0 = degp[:NP].reshape(NP, 1)
    d1 = degp[NP:].reshape(NP, 1)
    xp, inv = _prescale(d0, d1, x)
    xp2 = xp.reshape(2 * NN, CHH)
    p1 = _prop(xp2, srcr, dstr, zrows)
    z2p = _dense(p1, p1, xp, xp, inv, W1, b1.reshape(1, -1), W2)
    p2 = _prop(z2p.reshape(2 * NN, CHH), srcr, dstr, zrows)
    out = _final(p2, p2, z2p, z2p, inv, b2.reshape(1, -1))
    return out


# speedup vs baseline: 2.6228x; 1.0293x over previous
import functools

import jax
import jax.numpy as jnp
from jax import lax
from jax.experimental import pallas as pl
from jax.experimental.pallas import tpu as pltpu
from jax.experimental.pallas import tpu_sc as plsc

NN = 10000
CH_F = 128
CHH = 64
NP = 10240
CHUNK = 128
NCHW = 160
NQ = 5
NCQ = NCHW // NQ
NB = 5
GA = 2
SD = NB - GA
ND = 8
NCHD = 80
EW = CHUNK * NCHW
EPAD = EW * 16
RPT = NP // 16
RST = NN // 16

_MESH = plsc.VectorSubcoreMesh(core_axis_name="c", subcore_axis_name="s")


def _deg_body(dstr, z1, out, idxd, ones, accd, dsem):
    c = lax.axis_index("c")
    s = lax.axis_index("s")
    wid = c * 16 + s
    pltpu.sync_copy(z1, accd.at[pl.ds(s * RPT, RPT)])
    pltpu.sync_copy(dstr.at[wid], idxd)
    for i in range(CHUNK // 16):
        ones[pl.ds(i * 16, 16)] = jnp.ones((16,), jnp.float32)
    plsc.subcore_barrier()

    for b in range(ND):
        pltpu.async_copy(ones, accd.at[idxd.at[b]], dsem, add=True)

    def step(j, carry):
        pltpu.make_async_copy(ones, accd.at[idxd.at[j]], dsem).wait()
        nj = j + ND

        @pl.when(nj < NCHD)
        def _():
            pltpu.async_copy(ones, accd.at[idxd.at[nj]], dsem, add=True)

        return carry

    lax.fori_loop(0, NCHD, step, 0)
    plsc.subcore_barrier()
    pltpu.sync_copy(accd.at[pl.ds(s * RPT, RPT)], out.at[pl.ds(c * NP + s * RPT, RPT)])


_deg = functools.partial(
    pl.kernel,
    out_type=jax.ShapeDtypeStruct((2 * NP,), jnp.float32),
    mesh=_MESH,
    scratch_types=[
        pltpu.VMEM((NCHD, CHUNK), jnp.int32),
        pltpu.VMEM((CHUNK,), jnp.float32),
        pltpu.VMEM_SHARED((NP,), jnp.float32),
        pltpu.SemaphoreType.DMA,
    ],
)(_deg_body)


def _prop_body(y, srcr, dstr, zrows, out, idxs, idxd, rows, ytab, acc, gsem, ssem):
    c = lax.axis_index("c")
    s = lax.axis_index("s")
    pltpu.sync_copy(zrows, acc.at[pl.ds(s * RPT, RPT)])
    pltpu.sync_copy(y.at[pl.ds(c * NN + s * RST, RST)], ytab.at[pl.ds(s * RST, RST)])
    plsc.subcore_barrier()

    for q in range(NQ):
        pltpu.sync_copy(srcr.at[s, pl.ds(q * NCQ, NCQ)], idxs)
        pltpu.sync_copy(dstr.at[s, pl.ds(q * NCQ, NCQ)], idxd)

        for b in range(GA):
            pltpu.async_copy(ytab.at[idxs.at[b]], rows.at[b], gsem)

        def step(j, carry):
            b = lax.rem(j, NB)
            pltpu.make_async_copy(ytab.at[idxs.at[j]], rows.at[b], gsem).wait()
            pltpu.async_copy(rows.at[b], acc.at[idxd.at[j]], ssem, add=True)

            @pl.when(j >= SD)
            def _():
                pltpu.make_async_copy(rows.at[0], acc.at[idxd.at[0]], ssem).wait()

            nj = j + GA

            @pl.when(nj < NCQ)
            def _():
                pltpu.async_copy(ytab.at[idxs.at[nj]], rows.at[lax.rem(nj, NB)], gsem)

            return carry

        lax.fori_loop(0, NCQ, step, 0)

        def drain(j, carry):
            pltpu.make_async_copy(rows.at[0], acc.at[idxd.at[0]], ssem).wait()
            return carry

        lax.fori_loop(0, SD, drain, 0)

    plsc.subcore_barrier()
    pltpu.sync_copy(acc.at[pl.ds(s * RPT, RPT)], out.at[c, pl.ds(s * RPT, RPT)])


_prop = functools.partial(
    pl.kernel,
    out_type=jax.ShapeDtypeStruct((2, NP, CHH), jnp.float32),
    mesh=_MESH,
    compiler_params=pltpu.CompilerParams(use_tc_tiling_on_sc=False),
    scratch_types=[
        pltpu.VMEM((NCQ, CHUNK), jnp.int32),
        pltpu.VMEM((NCQ, CHUNK), jnp.int32),
        pltpu.VMEM((NB, CHUNK, CHH), jnp.float32),
        pltpu.VMEM_SHARED((NN, CHH), jnp.float32),
        pltpu.VMEM_SHARED((NP, CHH), jnp.float32),
        pltpu.SemaphoreType.DMA,
        pltpu.SemaphoreType.DMA,
    ],
)(_prop_body)


BR = 1000


def _prescale_body(d0, d1, x, xp, inv):
    d = d0[...] + d1[...] + 1.0
    r = lax.rsqrt(d)
    v = x[...] * r
    xp[0] = v[:, :CHH]
    xp[1] = v[:, CHH:]
    inv[...] = r


_prescale = pl.pallas_call(
    _prescale_body,
    grid=(NN // BR,),
    in_specs=[
        pl.BlockSpec((BR, 1), lambda i: (i, 0)),
        pl.BlockSpec((BR, 1), lambda i: (i, 0)),
        pl.BlockSpec((BR, CH_F), lambda i: (i, 0)),
    ],
    out_specs=[
        pl.BlockSpec((2, BR, CHH), lambda i: (0, i, 0)),
        pl.BlockSpec((BR, 1), lambda i: (i, 0)),
    ],
    out_shape=[
        jax.ShapeDtypeStruct((2, NN, CHH), jnp.float32),
        jax.ShapeDtypeStruct((NN, 1), jnp.float32),
    ],
)


def _dense_body(pa, pb, xa, xb, inv, w1, b1, w2, out):
    p = jnp.concatenate([pa[0], pb[0]], axis=1)
    xpv = jnp.concatenate([xa[0], xb[0]], axis=1)
    z1 = inv[...] * (p + xpv)
    h = jnp.dot(z1, w1[...], preferred_element_type=jnp.float32) + b1[...]
    h = jnp.maximum(h, 0.0)
    z2 = jnp.dot(h, w2[...], preferred_element_type=jnp.float32) * inv[...]
    out[0] = z2[:, :CHH]
    out[1] = z2[:, CHH:]


_dense = pl.pallas_call(
    _dense_body,
    grid=(NN // BR,),
    in_specs=[
        pl.BlockSpec((1, BR, CHH), lambda i: (0, i, 0)),
        pl.BlockSpec((1, BR, CHH), lambda i: (1, i, 0)),
        pl.BlockSpec((1, BR, CHH), lambda i: (0, i, 0)),
        pl.BlockSpec((1, BR, CHH), lambda i: (1, i, 0)),
        pl.BlockSpec((BR, 1), lambda i: (i, 0)),
        pl.BlockSpec((CH_F, 2 * CH_F), lambda i: (0, 0)),
        pl.BlockSpec((1, 2 * CH_F), lambda i: (0, 0)),
        pl.BlockSpec((2 * CH_F, CH_F), lambda i: (0, 0)),
    ],
    out_specs=pl.BlockSpec((2, BR, CHH), lambda i: (0, i, 0)),
    out_shape=jax.ShapeDtypeStruct((2, NN, CHH), jnp.float32),
)


def _final_body(pa, pb, za, zb, inv, b2, out):
    p = jnp.concatenate([pa[0], pb[0]], axis=1)
    z = jnp.concatenate([za[0], zb[0]], axis=1)
    out[...] = inv[...] * (p + z) + b2[...]


_final = pl.pallas_call(
    _final_body,
    grid=(NN // BR,),
    in_specs=[
        pl.BlockSpec((1, BR, CHH), lambda i: (0, i, 0)),
        pl.BlockSpec((1, BR, CHH), lambda i: (1, i, 0)),
        pl.BlockSpec((1, BR, CHH), lambda i: (0, i, 0)),
        pl.BlockSpec((1, BR, CHH), lambda i: (1, i, 0)),
        pl.BlockSpec((BR, 1), lambda i: (i, 0)),
        pl.BlockSpec((1, CH_F), lambda i: (0, 0)),
    ],
    out_specs=pl.BlockSpec((BR, CH_F), lambda i: (i, 0)),
    out_shape=jax.ShapeDtypeStruct((NN, CH_F), jnp.float32),
)


def kernel(x, edge_index, W1, b1, W2, b2):
    ei = edge_index.astype(jnp.int32)
    npad = EPAD - ei.shape[1]
    src = jnp.concatenate([ei[0], jnp.zeros((npad,), jnp.int32)])
    dst = jnp.concatenate([ei[1], jnp.full((npad,), NN, jnp.int32)])
    srcr = src.reshape(16, NCHW, CHUNK)
    dstr = dst.reshape(16, NCHW, CHUNK)
    dstr32 = dst.reshape(32, NCHD, CHUNK)
    zrows = jnp.zeros((RPT, CHH), jnp.float32)
    z1 = jnp.zeros((RPT,), jnp.float32)

    degp = _deg(dstr32, z1)
    d0 = degp[:NP].reshape(NP, 1)
    d1 = degp[NP:].reshape(NP, 1)
    xp, inv = _prescale(d0, d1, x)
    xp2 = xp.reshape(2 * NN, CHH)
    p1 = _prop(xp2, srcr, dstr, zrows)
    z2p = _dense(p1, p1, xp, xp, inv, W1, b1.reshape(1, -1), W2)
    p2 = _prop(z2p.reshape(2 * NN, CHH), srcr, dstr, zrows)
    out = _final(p2, p2, z2p, z2p, inv, b2.reshape(1, -1))
    return out


# speedup vs baseline: 2.6251x; 1.0009x over previous
import functools

import jax
import jax.numpy as jnp
from jax import lax
from jax.experimental import pallas as pl
from jax.experimental.pallas import tpu as pltpu
from jax.experimental.pallas import tpu_sc as plsc

NN = 10000
CH_F = 128
CHH = 64
NP = 10240
CHUNK = 128
NCHW = 160
NQ = 5
NCQ = NCHW // NQ
NB = 5
GA = 3
SD = NB - GA
ND = 8
NCHD = 80
EW = CHUNK * NCHW
EPAD = EW * 16
RPT = NP // 16
RST = NN // 16

_MESH = plsc.VectorSubcoreMesh(core_axis_name="c", subcore_axis_name="s")


def _deg_body(dstr, z1, out, idxd, ones, accd, dsem):
    c = lax.axis_index("c")
    s = lax.axis_index("s")
    wid = c * 16 + s
    pltpu.sync_copy(z1, accd.at[pl.ds(s * RPT, RPT)])
    pltpu.sync_copy(dstr.at[wid], idxd)
    for i in range(CHUNK // 16):
        ones[pl.ds(i * 16, 16)] = jnp.ones((16,), jnp.float32)
    plsc.subcore_barrier()

    for b in range(ND):
        pltpu.async_copy(ones, accd.at[idxd.at[b]], dsem, add=True)

    def step(j, carry):
        pltpu.make_async_copy(ones, accd.at[idxd.at[j]], dsem).wait()
        nj = j + ND

        @pl.when(nj < NCHD)
        def _():
            pltpu.async_copy(ones, accd.at[idxd.at[nj]], dsem, add=True)

        return carry

    lax.fori_loop(0, NCHD, step, 0)
    plsc.subcore_barrier()
    pltpu.sync_copy(accd.at[pl.ds(s * RPT, RPT)], out.at[pl.ds(c * NP + s * RPT, RPT)])


_deg = functools.partial(
    pl.kernel,
    out_type=jax.ShapeDtypeStruct((2 * NP,), jnp.float32),
    mesh=_MESH,
    scratch_types=[
        pltpu.VMEM((NCHD, CHUNK), jnp.int32),
        pltpu.VMEM((CHUNK,), jnp.float32),
        pltpu.VMEM_SHARED((NP,), jnp.float32),
        pltpu.SemaphoreType.DMA,
    ],
)(_deg_body)


def _prop_body(y, srcr, dstr, zrows, out, idxs, idxd, rows, ytab, acc, gsem, ssem):
    c = lax.axis_index("c")
    s = lax.axis_index("s")
    pltpu.sync_copy(zrows, acc.at[pl.ds(s * RPT, RPT)])
    pltpu.sync_copy(y.at[pl.ds(c * NN + s * RST, RST)], ytab.at[pl.ds(s * RST, RST)])
    plsc.subcore_barrier()

    for q in range(NQ):
        pltpu.sync_copy(srcr.at[s, pl.ds(q * NCQ, NCQ)], idxs)
        pltpu.sync_copy(dstr.at[s, pl.ds(q * NCQ, NCQ)], idxd)

        for b in range(GA):
            pltpu.async_copy(ytab.at[idxs.at[b]], rows.at[b], gsem)

        def step(j, carry):
            b = lax.rem(j, NB)
            pltpu.make_async_copy(ytab.at[idxs.at[j]], rows.at[b], gsem).wait()
            pltpu.async_copy(rows.at[b], acc.at[idxd.at[j]], ssem, add=True)

            @pl.when(j >= SD)
            def _():
                pltpu.make_async_copy(rows.at[0], acc.at[idxd.at[0]], ssem).wait()

            nj = j + GA

            @pl.when(nj < NCQ)
            def _():
                pltpu.async_copy(ytab.at[idxs.at[nj]], rows.at[lax.rem(nj, NB)], gsem)

            return carry

        lax.fori_loop(0, NCQ, step, 0)

        def drain(j, carry):
            pltpu.make_async_copy(rows.at[0], acc.at[idxd.at[0]], ssem).wait()
            return carry

        lax.fori_loop(0, SD, drain, 0)

    plsc.subcore_barrier()
    pltpu.sync_copy(acc.at[pl.ds(s * RPT, RPT)], out.at[c, pl.ds(s * RPT, RPT)])


_prop = functools.partial(
    pl.kernel,
    out_type=jax.ShapeDtypeStruct((2, NP, CHH), jnp.float32),
    mesh=_MESH,
    compiler_params=pltpu.CompilerParams(use_tc_tiling_on_sc=False),
    scratch_types=[
        pltpu.VMEM((NCQ, CHUNK), jnp.int32),
        pltpu.VMEM((NCQ, CHUNK), jnp.int32),
        pltpu.VMEM((NB, CHUNK, CHH), jnp.float32),
        pltpu.VMEM_SHARED((NN, CHH), jnp.float32),
        pltpu.VMEM_SHARED((NP, CHH), jnp.float32),
        pltpu.SemaphoreType.DMA,
        pltpu.SemaphoreType.DMA,
    ],
)(_prop_body)


BR = 1000


def _prescale_body(d0, d1, x, xp, inv):
    d = d0[...] + d1[...] + 1.0
    r = lax.rsqrt(d)
    v = x[...] * r
    xp[0] = v[:, :CHH]
    xp[1] = v[:, CHH:]
    inv[...] = r


_prescale = pl.pallas_call(
    _prescale_body,
    grid=(NN // BR,),
    in_specs=[
        pl.BlockSpec((BR, 1), lambda i: (i, 0)),
        pl.BlockSpec((BR, 1), lambda i: (i, 0)),
        pl.BlockSpec((BR, CH_F), lambda i: (i, 0)),
    ],
    out_specs=[
        pl.BlockSpec((2, BR, CHH), lambda i: (0, i, 0)),
        pl.BlockSpec((BR, 1), lambda i: (i, 0)),
    ],
    out_shape=[
        jax.ShapeDtypeStruct((2, NN, CHH), jnp.float32),
        jax.ShapeDtypeStruct((NN, 1), jnp.float32),
    ],
)


def _dense_body(pa, pb, xa, xb, inv, w1, b1, w2, out):
    p = jnp.concatenate([pa[0], pb[0]], axis=1)
    xpv = jnp.concatenate([xa[0], xb[0]], axis=1)
    z1 = inv[...] * (p + xpv)
    h = jnp.dot(z1, w1[...], preferred_element_type=jnp.float32) + b1[...]
    h = jnp.maximum(h, 0.0)
    z2 = jnp.dot(h, w2[...], preferred_element_type=jnp.float32) * inv[...]
    out[0] = z2[:, :CHH]
    out[1] = z2[:, CHH:]


_dense = pl.pallas_call(
    _dense_body,
    grid=(NN // BR,),
    in_specs=[
        pl.BlockSpec((1, BR, CHH), lambda i: (0, i, 0)),
        pl.BlockSpec((1, BR, CHH), lambda i: (1, i, 0)),
        pl.BlockSpec((1, BR, CHH), lambda i: (0, i, 0)),
        pl.BlockSpec((1, BR, CHH), lambda i: (1, i, 0)),
        pl.BlockSpec((BR, 1), lambda i: (i, 0)),
        pl.BlockSpec((CH_F, 2 * CH_F), lambda i: (0, 0)),
        pl.BlockSpec((1, 2 * CH_F), lambda i: (0, 0)),
        pl.BlockSpec((2 * CH_F, CH_F), lambda i: (0, 0)),
    ],
    out_specs=pl.BlockSpec((2, BR, CHH), lambda i: (0, i, 0)),
    out_shape=jax.ShapeDtypeStruct((2, NN, CHH), jnp.float32),
)


def _final_body(pa, pb, za, zb, inv, b2, out):
    p = jnp.concatenate([pa[0], pb[0]], axis=1)
    z = jnp.concatenate([za[0], zb[0]], axis=1)
    out[...] = inv[...] * (p + z) + b2[...]


_final = pl.pallas_call(
    _final_body,
    grid=(NN // BR,),
    in_specs=[
        pl.BlockSpec((1, BR, CHH), lambda i: (0, i, 0)),
        pl.BlockSpec((1, BR, CHH), lambda i: (1, i, 0)),
        pl.BlockSpec((1, BR, CHH), lambda i: (0, i, 0)),
        pl.BlockSpec((1, BR, CHH), lambda i: (1, i, 0)),
        pl.BlockSpec((BR, 1), lambda i: (i, 0)),
        pl.BlockSpec((1, CH_F), lambda i: (0, 0)),
    ],
    out_specs=pl.BlockSpec((BR, CH_F), lambda i: (i, 0)),
    out_shape=jax.ShapeDtypeStruct((NN, CH_F), jnp.float32),
)


def kernel(x, edge_index, W1, b1, W2, b2):
    ei = edge_index.astype(jnp.int32)
    npad = EPAD - ei.shape[1]
    src = jnp.concatenate([ei[0], jnp.zeros((npad,), jnp.int32)])
    dst = jnp.concatenate([ei[1], jnp.full((npad,), NN, jnp.int32)])
    srcr = src.reshape(16, NCHW, CHUNK)
    dstr = dst.reshape(16, NCHW, CHUNK)
    dstr32 = dst.reshape(32, NCHD, CHUNK)
    zrows = jnp.zeros((RPT, CHH), jnp.float32)
    z1 = jnp.zeros((RPT,), jnp.float32)

    degp = _deg(dstr32, z1)
    d0 = degp[:NP].reshape(NP, 1)
    d1 = degp[NP:].reshape(NP, 1)
    xp, inv = _prescale(d0, d1, x)
    xp2 = xp.reshape(2 * NN, CHH)
    p1 = _prop(xp2, srcr, dstr, zrows)
    z2p = _dense(p1, p1, xp, xp, inv, W1, b1.reshape(1, -1), W2)
    p2 = _prop(z2p.reshape(2 * NN, CHH), srcr, dstr, zrows)
    out = _final(p2, p2, z2p, z2p, inv, b2.reshape(1, -1))
    return out
